# baseline (speedup 1.0000x reference)
import jax
import jax.numpy as jnp
from jax import lax
from jax.experimental import pallas as pl
from jax.experimental.pallas import tpu as pltpu

K = 8192
D = 256
ROWS = 4608
RB = 512
KBLK = 2048


def _argmin_body(x_ref, w_ref, sx_ref, sw_ref, idx_ref, best_ref, bidx_ref):
    kb = pl.program_id(1)
    x = x_ref[...]
    w = w_ref[pl.ds(kb * KBLK, KBLK), :]
    mm = lax.dot_general(x, w, (((1,), (1,)), ((), ())),
                         preferred_element_type=jnp.float32)
    d = (sx_ref[...] + sw_ref[:, pl.ds(kb * KBLK, KBLK)]) - 2.0 * mm
    m = jnp.min(d, axis=1, keepdims=True)
    ii = lax.broadcasted_iota(jnp.int32, (RB, KBLK), 1) + kb * KBLK
    cand = jnp.where(d == m, ii, jnp.int32(2**30))
    imin = jnp.min(cand, axis=1, keepdims=True)

    @pl.when(kb == 0)
    def _():
        best_ref[...] = m
        bidx_ref[...] = imin

    @pl.when(kb > 0)
    def _():
        b = best_ref[...]
        bi = bidx_ref[...]
        better = m < b
        best_ref[...] = jnp.where(better, m, b)
        bidx_ref[...] = jnp.where(better, imin, bi)

    @pl.when(kb == (K // KBLK) - 1)
    def _():
        idx_ref[...] = bidx_ref[...]


def _argmin_indices(xn, wn, sx, sw):
    return pl.pallas_call(
        _argmin_body,
        grid=(ROWS // RB, K // KBLK),
        in_specs=[
            pl.BlockSpec((RB, D), lambda r, kb: (r, 0)),
            pl.BlockSpec((K, D), lambda r, kb: (0, 0)),
            pl.BlockSpec((RB, 1), lambda r, kb: (r, 0)),
            pl.BlockSpec((1, K), lambda r, kb: (0, 0)),
        ],
        out_specs=pl.BlockSpec((RB, 1), lambda r, kb: (r, 0)),
        out_shape=jax.ShapeDtypeStruct((ROWS, 1), jnp.int32),
        scratch_shapes=[
            pltpu.VMEM((RB, 1), jnp.float32),
            pltpu.VMEM((RB, 1), jnp.int32),
        ],
    )(xn, wn, sx, sw)


def kernel(z, W):
    inputs = jnp.transpose(z, (0, 2, 3, 1))
    input_shape = inputs.shape
    flat_x = inputs.reshape(-1, D)
    nx = jnp.linalg.norm(flat_x, axis=1, keepdims=True)
    xn = flat_x / jnp.clip(nx, 1e-12)
    nw = jnp.linalg.norm(W, axis=1, keepdims=True)
    wn = W / jnp.clip(nw, 1e-12)
    sx = jnp.sum(xn ** 2, axis=1, keepdims=True)
    sw = jnp.sum(wn ** 2, axis=1)[None, :]

    idx = _argmin_indices(xn, wn, sx, sw)[:, 0]

    encodings = jax.nn.one_hot(idx, K, dtype=jnp.float32)
    quantized = W[idx].reshape(input_shape)
    e_latent_loss = jnp.mean((quantized - inputs) ** 2)
    loss = 0.25 * e_latent_loss
    quantized_st = inputs + (quantized - inputs)
    quantized_out = jnp.transpose(quantized_st, (0, 3, 1, 2))
    avg_probs = jnp.mean(encodings, axis=0)
    perplexity = jnp.exp(-jnp.sum(avg_probs * jnp.log(avg_probs + 1e-10)))
    return (quantized_out, loss, perplexity, encodings)


# speedup vs baseline: 1.0124x; 1.0124x over previous
import functools

import jax
import jax.numpy as jnp
from jax import lax
from jax.experimental import pallas as pl
from jax.experimental.pallas import tpu as pltpu
from jax.experimental.pallas import tpu_sc as plsc

K = 8192
D = 256
ROWS = 4608
RB = 512
KBLK = 2048
RB2 = 512
KB2 = 2048

NW = 32
BPW = ROWS // NW


def _argmin_body(x_ref, w_ref, sx_ref, sw_ref, idx_ref, best_ref, bidx_ref):
    kb = pl.program_id(1)
    x = x_ref[...]
    w = w_ref[pl.ds(kb * KBLK, KBLK), :]
    mm = lax.dot_general(x, w, (((1,), (1,)), ((), ())),
                         preferred_element_type=jnp.float32)
    d = (sx_ref[...] + sw_ref[:, pl.ds(kb * KBLK, KBLK)]) - 2.0 * mm
    m = jnp.min(d, axis=1, keepdims=True)
    ii = lax.broadcasted_iota(jnp.int32, (RB, KBLK), 1) + kb * KBLK
    cand = jnp.where(d == m, ii, jnp.int32(2**30))
    imin = jnp.min(cand, axis=1, keepdims=True)

    @pl.when(kb == 0)
    def _():
        best_ref[...] = m
        bidx_ref[...] = imin

    @pl.when(kb > 0)
    def _():
        b = best_ref[...]
        bi = bidx_ref[...]
        better = m < b
        best_ref[...] = jnp.where(better, m, b)
        bidx_ref[...] = jnp.where(better, imin, bi)

    @pl.when(kb == (K // KBLK) - 1)
    def _():
        idx_ref[...] = bidx_ref[...]


def _argmin_indices(xn, wn, sx, sw):
    return pl.pallas_call(
        _argmin_body,
        grid=(ROWS // RB, K // KBLK),
        in_specs=[
            pl.BlockSpec((RB, D), lambda r, kb: (r, 0)),
            pl.BlockSpec((K, D), lambda r, kb: (0, 0)),
            pl.BlockSpec((RB, 1), lambda r, kb: (r, 0)),
            pl.BlockSpec((1, K), lambda r, kb: (0, 0)),
        ],
        out_specs=pl.BlockSpec((RB, 1), lambda r, kb: (r, 0)),
        out_shape=jax.ShapeDtypeStruct((ROWS, 1), jnp.int32),
        scratch_shapes=[
            pltpu.VMEM((RB, 1), jnp.float32),
            pltpu.VMEM((RB, 1), jnp.int32),
        ],
    )(xn, wn, sx, sw)


def _sc_gather(table, idx):
    mesh = plsc.VectorSubcoreMesh(core_axis_name="c", subcore_axis_name="s",
                                  num_cores=2, num_subcores=16)

    @functools.partial(
        pl.kernel, mesh=mesh,
        out_type=jax.ShapeDtypeStruct((ROWS, D), jnp.float32),
        scratch_types=[
            pltpu.VMEM((BPW,), jnp.int32),
            pltpu.VMEM((BPW, D), jnp.float32),
            pltpu.SemaphoreType.DMA,
        ],
    )
    def gather_kernel(table_hbm, idx_hbm, out_hbm, idx_v, rows_v, sem):
        wid = lax.axis_index("s") * 2 + lax.axis_index("c")
        base = wid * BPW
        pltpu.sync_copy(idx_hbm.at[pl.ds(base, BPW)], idx_v)
        pltpu.async_copy(table_hbm.at[idx_v], rows_v, sem).wait()
        pltpu.sync_copy(rows_v, out_hbm.at[pl.ds(base, BPW)])

    return gather_kernel(table, idx)


def _enc_body(idx_ref, enc_ref, counts_ref, perp_ref):
    r = pl.program_id(0)
    kb = pl.program_id(1)
    idxb = idx_ref[...]
    ii = lax.broadcasted_iota(jnp.int32, (RB2, KB2), 1) + kb * KB2
    enc = jnp.where(ii == idxb, 1.0, 0.0).astype(jnp.float32)
    enc_ref[...] = enc
    colsum = jnp.sum(enc, axis=0, keepdims=True)

    @pl.when(r == 0)
    def _():
        counts_ref[:, pl.ds(kb * KB2, KB2)] = colsum

    @pl.when(r > 0)
    def _():
        counts_ref[:, pl.ds(kb * KB2, KB2)] = (
            counts_ref[:, pl.ds(kb * KB2, KB2)] + colsum)

    last = (r == (ROWS // RB2) - 1) & (kb == (K // KB2) - 1)

    @pl.when(last)
    def _():
        p = counts_ref[...] * (1.0 / ROWS)
        ent = jnp.sum(p * jnp.log(p + 1e-10), axis=1, keepdims=True)
        perp_ref[...] = jnp.exp(-ent)


def _encodings_counts(idx2):
    return pl.pallas_call(
        _enc_body,
        grid=(ROWS // RB2, K // KB2),
        in_specs=[pl.BlockSpec((RB2, 1), lambda r, kb: (r, 0))],
        out_specs=[
            pl.BlockSpec((RB2, KB2), lambda r, kb: (r, kb)),
            pl.BlockSpec((1, K), lambda r, kb: (0, 0)),
            pl.BlockSpec((1, 1), lambda r, kb: (0, 0)),
        ],
        out_shape=[
            jax.ShapeDtypeStruct((ROWS, K), jnp.float32),
            jax.ShapeDtypeStruct((1, K), jnp.float32),
            jax.ShapeDtypeStruct((1, 1), jnp.float32),
        ],
    )(idx2)


def _st_loss_body(q_ref, x_ref, qst_ref, loss_ref):
    q = q_ref[...]
    x = x_ref[...]
    diff = q - x
    qst_ref[...] = x + diff
    s = jnp.sum(diff * diff, axis=1, keepdims=True)
    s0 = jnp.sum(s, axis=0, keepdims=True)
    loss_ref[...] = 0.25 * (s0 * (1.0 / (ROWS * D)))


def _st_loss(q, flat_x):
    return pl.pallas_call(
        _st_loss_body,
        grid=(1,),
        in_specs=[
            pl.BlockSpec((ROWS, D), lambda i: (0, 0)),
            pl.BlockSpec((ROWS, D), lambda i: (0, 0)),
        ],
        out_specs=[
            pl.BlockSpec((ROWS, D), lambda i: (0, 0)),
            pl.BlockSpec((1, 1), lambda i: (0, 0)),
        ],
        out_shape=[
            jax.ShapeDtypeStruct((ROWS, D), jnp.float32),
            jax.ShapeDtypeStruct((1, 1), jnp.float32),
        ],
    )(q, flat_x)


def kernel(z, W):
    inputs = jnp.transpose(z, (0, 2, 3, 1))
    input_shape = inputs.shape
    flat_x = inputs.reshape(-1, D)
    nx = jnp.linalg.norm(flat_x, axis=1, keepdims=True)
    xn = flat_x / jnp.clip(nx, 1e-12)
    nw = jnp.linalg.norm(W, axis=1, keepdims=True)
    wn = W / jnp.clip(nw, 1e-12)
    sx = jnp.sum(xn ** 2, axis=1, keepdims=True)
    sw = jnp.sum(wn ** 2, axis=1)[None, :]

    idx2 = _argmin_indices(xn, wn, sx, sw)
    idx = idx2.reshape(ROWS)

    q = _sc_gather(W, idx)
    encodings, _counts, perp = _encodings_counts(idx2)
    qst, loss = _st_loss(q, flat_x)

    quantized_out = jnp.transpose(qst.reshape(input_shape), (0, 3, 1, 2))
    return (quantized_out, loss[0, 0], perp[0, 0], encodings)
